# baseline (speedup 1.0000x reference)
import functools

import jax
import jax.numpy as jnp
from jax import lax
from jax.experimental import pallas as pl
from jax.experimental.pallas import tpu as pltpu
from jax.experimental.pallas import tpu_sc as plsc

D = 1024
FF = 4096
E = 8
K = 2

BLK = 512
FFT = 512
J = FF // FFT
T = 4096
A = T * K
NB = A // BLK + E
P = NB * BLK

NC, NS, L = 2, 16, 16
NW = NC * NS

TB = 1024


def _router_body(x_ref, rw_ref, rb_ref, e_ref, w_ref):
    logits = jnp.dot(x_ref[...], rw_ref[...], preferred_element_type=jnp.float32)
    logits = logits + rb_ref[...]
    idx8 = lax.broadcasted_iota(jnp.int32, (TB, E), 1)
    m0 = jnp.max(logits, axis=-1, keepdims=True)
    e0 = jnp.min(jnp.where(logits == m0, idx8, E), axis=-1, keepdims=True)
    masked = jnp.where(idx8 == e0, -jnp.inf, logits)
    m1 = jnp.max(masked, axis=-1, keepdims=True)
    e1 = jnp.min(jnp.where(masked == m1, idx8, E), axis=-1, keepdims=True)
    w0 = jax.nn.sigmoid(m0 - m1)
    e_ref[...] = jnp.concatenate([e0, e1], axis=1)
    w_ref[...] = jnp.concatenate([w0, 1.0 - w0], axis=1)


def _router(x_flat, rw, rb2):
    return pl.pallas_call(
        _router_body,
        grid=(T // TB,),
        in_specs=[
            pl.BlockSpec((TB, D), lambda i: (i, 0)),
            pl.BlockSpec((D, E), lambda i: (0, 0)),
            pl.BlockSpec((1, E), lambda i: (0, 0)),
        ],
        out_specs=[
            pl.BlockSpec((TB, K), lambda i: (i, 0)),
            pl.BlockSpec((TB, K), lambda i: (i, 0)),
        ],
        out_shape=[
            jax.ShapeDtypeStruct((T, K), jnp.int32),
            jax.ShapeDtypeStruct((T, K), jnp.float32),
        ],
    )(x_flat, rw, rb2)


_G_SLOTS = P // NW
_G_CH = 64


def _dispatch_body(src_hbm, x_hbm, xg_hbm, idx_v, rows_v, sem):
    wid = lax.axis_index("s") * NC + lax.axis_index("c")
    base = wid * _G_SLOTS
    pltpu.sync_copy(src_hbm.at[pl.ds(base, _G_SLOTS)], idx_v)
    for k in range(_G_SLOTS // _G_CH):
        pltpu.async_copy(
            x_hbm.at[idx_v.at[pl.ds(k * _G_CH, _G_CH)]], rows_v, sem
        ).wait()
        pltpu.sync_copy(rows_v, xg_hbm.at[pl.ds(base + k * _G_CH, _G_CH)])


def _dispatch(src_token, x_flat):
    mesh = plsc.VectorSubcoreMesh(core_axis_name="c", subcore_axis_name="s")
    return pl.kernel(
        _dispatch_body,
        out_type=jax.ShapeDtypeStruct((P, D), jnp.float32),
        mesh=mesh,
        scratch_types=[
            pltpu.VMEM((_G_SLOTS,), jnp.int32),
            pltpu.VMEM((_G_CH, D), jnp.float32),
            pltpu.SemaphoreType.DMA,
        ],
    )(src_token, x_flat)


def _ffn_body(be, bv, xg_ref, w1_ref, b1_ref, w2_ref, b2_ref, gw_ref,
              y_ref, acc_ref):
    b = pl.program_id(0)
    j = pl.program_id(1)
    valid = bv[b] == 1

    @pl.when(valid)
    def _():
        xb = xg_ref[...].astype(jnp.bfloat16)
        h = jnp.dot(xb, w1_ref[0], preferred_element_type=jnp.float32)
        h = h + b1_ref[0]
        h = 0.5 * h * (1.0 + lax.erf(h * (2.0 ** -0.5)))
        part = jnp.dot(h.astype(jnp.bfloat16), w2_ref[0],
                       preferred_element_type=jnp.float32)

        @pl.when(j == 0)
        def _():
            acc_ref[...] = part

        @pl.when(j > 0)
        def _():
            acc_ref[...] += part

    @pl.when(valid & (j == J - 1))
    def _():
        y_ref[...] = (acc_ref[...] + b2_ref[0]) * gw_ref[...]


def _ffn(block_e, block_valid, xg, w1, b1, w2, b2, gw2):
    grid_spec = pltpu.PrefetchScalarGridSpec(
        num_scalar_prefetch=2,
        grid=(NB, J),
        in_specs=[
            pl.BlockSpec((BLK, D), lambda b, j, be, bv: (jnp.where(bv[b] == 1, b, 0), 0)),
            pl.BlockSpec((1, D, FFT), lambda b, j, be, bv: (be[b], 0, jnp.where(bv[b] == 1, j, 0))),
            pl.BlockSpec((1, 1, FFT), lambda b, j, be, bv: (be[b], 0, jnp.where(bv[b] == 1, j, 0))),
            pl.BlockSpec((1, FFT, D), lambda b, j, be, bv: (be[b], jnp.where(bv[b] == 1, j, 0), 0)),
            pl.BlockSpec((1, 1, D), lambda b, j, be, bv: (be[b], 0, 0)),
            pl.BlockSpec((BLK, 1), lambda b, j, be, bv: (jnp.where(bv[b] == 1, b, 0), 0)),
        ],
        out_specs=pl.BlockSpec((BLK, D), lambda b, j, be, bv: (b, 0)),
        scratch_shapes=[pltpu.VMEM((BLK, D), jnp.float32)],
    )
    return pl.pallas_call(
        _ffn_body,
        grid_spec=grid_spec,
        out_shape=jax.ShapeDtypeStruct((P, D), jnp.float32),
        compiler_params=pltpu.CompilerParams(
            dimension_semantics=("arbitrary", "arbitrary")),
    )(block_e, block_valid, xg, w1.astype(jnp.bfloat16), b1.reshape(E, 1, FF),
      w2.astype(jnp.bfloat16), b2.reshape(E, 1, D), gw2)


_C_TOK = T // NW
_C_CH = 16


def _combine_body(y_hbm, pos_hbm, out_hbm, pos_v, rows_v, out_v, sem):
    wid = lax.axis_index("s") * NC + lax.axis_index("c")
    pltpu.sync_copy(pos_hbm.at[pl.ds(wid * K * _C_TOK, K * _C_TOK)], pos_v)
    for k in range(_C_TOK // _C_CH):
        pltpu.async_copy(
            y_hbm.at[pos_v.at[pl.ds(k * K * _C_CH, K * _C_CH)]], rows_v, sem
        ).wait()

        def body(i, _):
            for dd in range(D // L):
                sl = pl.ds(dd * L, L)
                out_v[i, sl] = rows_v[2 * i, sl] + rows_v[2 * i + 1, sl]
            return 0

        lax.fori_loop(0, _C_CH, body, 0)
        pltpu.sync_copy(out_v, out_hbm.at[pl.ds(wid * _C_TOK + k * _C_CH, _C_CH)])


def _combine(y, pos_flat):
    mesh = plsc.VectorSubcoreMesh(core_axis_name="c", subcore_axis_name="s")
    return pl.kernel(
        _combine_body,
        out_type=jax.ShapeDtypeStruct((T, D), jnp.float32),
        mesh=mesh,
        scratch_types=[
            pltpu.VMEM((K * _C_TOK,), jnp.int32),
            pltpu.VMEM((K * _C_CH, D), jnp.float32),
            pltpu.VMEM((_C_CH, D), jnp.float32),
            pltpu.SemaphoreType.DMA,
        ],
    )(y, pos_flat)


def kernel(x, router_w, router_b, w1, b1, w2, b2):
    B, S, _ = x.shape
    x_flat = x.reshape(T, D)

    e01, w01 = _router(x_flat, router_w, router_b.reshape(1, E))

    e_flat = e01.reshape(A)
    w_flat = w01.reshape(A)
    onehot = (e_flat[:, None] == jnp.arange(E, dtype=jnp.int32)[None, :]).astype(jnp.int32)
    incl = jnp.cumsum(onehot, axis=0)
    counts = incl[-1]
    rank = jnp.sum((incl - onehot) * onehot, axis=-1)
    nb = (counts + BLK - 1) // BLK
    cum_nb = jnp.cumsum(nb)
    blk_start = cum_nb - nb
    pos = blk_start[e_flat] * BLK + rank
    src_token = jnp.zeros((P,), jnp.int32).at[pos].set(
        jnp.arange(A, dtype=jnp.int32) // K)
    gw = jnp.zeros((P,), jnp.float32).at[pos].set(w_flat)
    bidx = jnp.arange(NB, dtype=jnp.int32)
    block_e = jnp.minimum(
        jnp.searchsorted(cum_nb, bidx, side="right"), E - 1).astype(jnp.int32)
    block_valid = (bidx < cum_nb[-1]).astype(jnp.int32)

    xg = _dispatch(src_token, x_flat)
    y = _ffn(block_e, block_valid, xg, w1, b1, w2, b2, gw.reshape(P, 1))
    out_flat = _combine(y, pos.astype(jnp.int32))
    return out_flat.reshape(B, S, D)


# speedup vs baseline: 1.1688x; 1.1688x over previous
import functools

import jax
import jax.numpy as jnp
from jax import lax
from jax.experimental import pallas as pl
from jax.experimental.pallas import tpu as pltpu
from jax.experimental.pallas import tpu_sc as plsc

D = 1024
FF = 4096
E = 8
K = 2

BLK = 512
FFT = 512
J = FF // FFT
T = 4096
A = T * K
NB = A // BLK + E
P = NB * BLK

NC, NS, L = 2, 16, 16
NW = NC * NS

TB = 1024


def _router_body(x_ref, rw_ref, rb_ref, e_ref, w_ref):
    logits = jnp.dot(x_ref[...], rw_ref[...], preferred_element_type=jnp.float32)
    logits = logits + rb_ref[...]
    idx8 = lax.broadcasted_iota(jnp.int32, (TB, E), 1)
    m0 = jnp.max(logits, axis=-1, keepdims=True)
    e0 = jnp.min(jnp.where(logits == m0, idx8, E), axis=-1, keepdims=True)
    masked = jnp.where(idx8 == e0, -jnp.inf, logits)
    m1 = jnp.max(masked, axis=-1, keepdims=True)
    e1 = jnp.min(jnp.where(masked == m1, idx8, E), axis=-1, keepdims=True)
    w0 = jax.nn.sigmoid(m0 - m1)
    e_ref[...] = jnp.concatenate([e0, e1], axis=1)
    w_ref[...] = jnp.concatenate([w0, 1.0 - w0], axis=1)


def _router(x_flat, rw, rb2):
    return pl.pallas_call(
        _router_body,
        grid=(T // TB,),
        in_specs=[
            pl.BlockSpec((TB, D), lambda i: (i, 0)),
            pl.BlockSpec((D, E), lambda i: (0, 0)),
            pl.BlockSpec((1, E), lambda i: (0, 0)),
        ],
        out_specs=[
            pl.BlockSpec((TB, K), lambda i: (i, 0)),
            pl.BlockSpec((TB, K), lambda i: (i, 0)),
        ],
        out_shape=[
            jax.ShapeDtypeStruct((T, K), jnp.int32),
            jax.ShapeDtypeStruct((T, K), jnp.float32),
        ],
    )(x_flat, rw, rb2)


_G_SLOTS = P // NW
_G_CH = 48
_G_N = _G_SLOTS // _G_CH


def _dispatch_body(src_hbm, x_hbm, xg_hbm, idx_v, rows0, rows1,
                   gs0, gs1, ss0, ss1):
    wid = lax.axis_index("s") * NC + lax.axis_index("c")
    base = wid * _G_SLOTS
    pltpu.sync_copy(src_hbm.at[pl.ds(base, _G_SLOTS)], idx_v)
    bufs, gsems, ssems = (rows0, rows1), (gs0, gs1), (ss0, ss1)
    g = [None] * _G_N
    s = [None] * _G_N
    g[0] = pltpu.async_copy(
        x_hbm.at[idx_v.at[pl.ds(0, _G_CH)]], bufs[0], gsems[0])
    for k in range(_G_N):
        cur = k & 1
        if k + 1 < _G_N:
            if k >= 1:
                s[k - 1].wait()
            g[k + 1] = pltpu.async_copy(
                x_hbm.at[idx_v.at[pl.ds((k + 1) * _G_CH, _G_CH)]],
                bufs[(k + 1) & 1], gsems[(k + 1) & 1])
        g[k].wait()
        s[k] = pltpu.async_copy(
            bufs[cur], xg_hbm.at[pl.ds(base + k * _G_CH, _G_CH)], ssems[cur])
    s[_G_N - 2].wait()
    s[_G_N - 1].wait()


def _dispatch(src_token, x_flat):
    mesh = plsc.VectorSubcoreMesh(core_axis_name="c", subcore_axis_name="s")
    return pl.kernel(
        _dispatch_body,
        out_type=jax.ShapeDtypeStruct((P, D), jnp.float32),
        mesh=mesh,
        scratch_types=[
            pltpu.VMEM((_G_SLOTS,), jnp.int32),
            pltpu.VMEM((_G_CH, D), jnp.float32),
            pltpu.VMEM((_G_CH, D), jnp.float32),
            pltpu.SemaphoreType.DMA,
            pltpu.SemaphoreType.DMA,
            pltpu.SemaphoreType.DMA,
            pltpu.SemaphoreType.DMA,
        ],
    )(src_token, x_flat)


def _ffn_body(be, bv, xg_ref, w1_ref, b1_ref, w2_ref, b2_ref, gw_ref,
              y_ref, acc_ref):
    b = pl.program_id(0)
    j = pl.program_id(1)
    valid = bv[b] == 1

    @pl.when(valid)
    def _():
        h = jnp.dot(xg_ref[...], w1_ref[0], preferred_element_type=jnp.float32)
        h = h + b1_ref[0]
        h = 0.5 * h * (1.0 + lax.erf(h * (2.0 ** -0.5)))
        part = jnp.dot(h, w2_ref[0], preferred_element_type=jnp.float32)

        @pl.when(j == 0)
        def _():
            acc_ref[...] = part

        @pl.when(j > 0)
        def _():
            acc_ref[...] += part

    @pl.when(valid & (j == J - 1))
    def _():
        y_ref[...] = (acc_ref[...] + b2_ref[0]) * gw_ref[...]


def _ffn(block_e, block_valid, xg, w1, b1, w2, b2, gw2):
    grid_spec = pltpu.PrefetchScalarGridSpec(
        num_scalar_prefetch=2,
        grid=(NB, J),
        in_specs=[
            pl.BlockSpec((BLK, D), lambda b, j, be, bv: (jnp.where(bv[b] == 1, b, 0), 0)),
            pl.BlockSpec((1, D, FFT), lambda b, j, be, bv: (be[b], 0, jnp.where(bv[b] == 1, j, 0))),
            pl.BlockSpec((1, 1, FFT), lambda b, j, be, bv: (be[b], 0, jnp.where(bv[b] == 1, j, 0))),
            pl.BlockSpec((1, FFT, D), lambda b, j, be, bv: (be[b], jnp.where(bv[b] == 1, j, 0), 0)),
            pl.BlockSpec((1, 1, D), lambda b, j, be, bv: (be[b], 0, 0)),
            pl.BlockSpec((BLK, 1), lambda b, j, be, bv: (jnp.where(bv[b] == 1, b, 0), 0)),
        ],
        out_specs=pl.BlockSpec((BLK, D), lambda b, j, be, bv: (b, 0)),
        scratch_shapes=[pltpu.VMEM((BLK, D), jnp.float32)],
    )
    return pl.pallas_call(
        _ffn_body,
        grid_spec=grid_spec,
        out_shape=jax.ShapeDtypeStruct((P, D), jnp.float32),
        compiler_params=pltpu.CompilerParams(
            dimension_semantics=("arbitrary", "arbitrary")),
    )(block_e, block_valid, xg, w1, b1.reshape(E, 1, FF), w2,
      b2.reshape(E, 1, D), gw2)


_C_TOK = T // NW
_C_CH = 16


_C_N = _C_TOK // _C_CH


def _combine_body(y_hbm, pos_hbm, out_hbm, pos_v, rows0, rows1, out0, out1,
                  gs0, gs1, ss0, ss1):
    wid = lax.axis_index("s") * NC + lax.axis_index("c")
    pltpu.sync_copy(pos_hbm.at[pl.ds(wid * K * _C_TOK, K * _C_TOK)], pos_v)
    rbufs, obufs = (rows0, rows1), (out0, out1)
    gsems, ssems = (gs0, gs1), (ss0, ss1)
    g = [None] * _C_N
    s = [None] * _C_N
    g[0] = pltpu.async_copy(
        y_hbm.at[pos_v.at[pl.ds(0, K * _C_CH)]], rbufs[0], gsems[0])
    for k in range(_C_N):
        cur = k & 1
        if k + 1 < _C_N:
            g[k + 1] = pltpu.async_copy(
                y_hbm.at[pos_v.at[pl.ds((k + 1) * K * _C_CH, K * _C_CH)]],
                rbufs[(k + 1) & 1], gsems[(k + 1) & 1])
        g[k].wait()
        if k >= 2:
            s[k - 2].wait()
        rows_v, out_v = rbufs[cur], obufs[cur]

        def body(i, _):
            for dd in range(D // L):
                sl = pl.ds(dd * L, L)
                out_v[i, sl] = rows_v[2 * i, sl] + rows_v[2 * i + 1, sl]
            return 0

        lax.fori_loop(0, _C_CH, body, 0)
        s[k] = pltpu.async_copy(
            out_v, out_hbm.at[pl.ds(wid * _C_TOK + k * _C_CH, _C_CH)],
            ssems[cur])
    s[_C_N - 2].wait()
    s[_C_N - 1].wait()


def _combine(y, pos_flat):
    mesh = plsc.VectorSubcoreMesh(core_axis_name="c", subcore_axis_name="s")
    return pl.kernel(
        _combine_body,
        out_type=jax.ShapeDtypeStruct((T, D), jnp.float32),
        mesh=mesh,
        scratch_types=[
            pltpu.VMEM((K * _C_TOK,), jnp.int32),
            pltpu.VMEM((K * _C_CH, D), jnp.float32),
            pltpu.VMEM((K * _C_CH, D), jnp.float32),
            pltpu.VMEM((_C_CH, D), jnp.float32),
            pltpu.VMEM((_C_CH, D), jnp.float32),
            pltpu.SemaphoreType.DMA,
            pltpu.SemaphoreType.DMA,
            pltpu.SemaphoreType.DMA,
            pltpu.SemaphoreType.DMA,
        ],
    )(y, pos_flat)


def kernel(x, router_w, router_b, w1, b1, w2, b2):
    B, S, _ = x.shape
    x_flat = x.reshape(T, D)

    e01, w01 = _router(x_flat, router_w, router_b.reshape(1, E))

    e_flat = e01.reshape(A)
    w_flat = w01.reshape(A)
    onehot = (e_flat[:, None] == jnp.arange(E, dtype=jnp.int32)[None, :]).astype(jnp.int32)
    incl = jnp.cumsum(onehot, axis=0)
    counts = incl[-1]
    rank = jnp.sum((incl - onehot) * onehot, axis=-1)
    nb = (counts + BLK - 1) // BLK
    cum_nb = jnp.cumsum(nb)
    blk_start = cum_nb - nb
    pos = blk_start[e_flat] * BLK + rank
    src_token = jnp.zeros((P,), jnp.int32).at[pos].set(
        jnp.arange(A, dtype=jnp.int32) // K)
    gw = jnp.zeros((P,), jnp.float32).at[pos].set(w_flat)
    bidx = jnp.arange(NB, dtype=jnp.int32)
    block_e = jnp.minimum(
        jnp.searchsorted(cum_nb, bidx, side="right"), E - 1).astype(jnp.int32)
    block_valid = (bidx < cum_nb[-1]).astype(jnp.int32)

    xg = _dispatch(src_token, x_flat)
    y = _ffn(block_e, block_valid, xg, w1, b1, w2, b2, gw.reshape(P, 1))
    out_flat = _combine(y, pos.astype(jnp.int32))
    return out_flat.reshape(B, S, D)
